# baseline (speedup 1.0000x reference)
import functools

import jax
import jax.numpy as jnp
from jax import lax
from jax.experimental import pallas as pl
from jax.experimental.pallas import tpu as pltpu
from jax.experimental.pallas import tpu_sc as plsc

B = 4
N = 2048
D = 256
K = 16
SCALE = 0.0625


_ROWS = 256


def _topk_body(xyz_blk_ref, xyz_all_ref, idx_ref, *, b):
    x = xyz_blk_ref[0]
    y = xyz_all_ref[0]
    sqx = jnp.sum(x * x, axis=1)
    sqy = jnp.sum(y * y, axis=1)
    dot = lax.dot_general(
        x, y, (((1,), (1,)), ((), ())),
        preferred_element_type=jnp.float32,
    )
    d2 = (sqx[:, None] + sqy[None, :]) - 2.0 * dot
    vals = jnp.sqrt(jnp.maximum(d2, 0.0))
    G = N // 4
    s = [vals[:, k * G:(k + 1) * G] for k in range(4)]
    iotaf = lax.broadcasted_iota(jnp.int32, (_ROWS, G), 1).astype(jnp.float32)
    ii = [iotaf + jnp.float32(k * G) for k in range(4)]

    def cmpex(p, q, lex):
        if lex:
            sw = (s[q] < s[p]) | ((s[q] == s[p]) & (ii[q] < ii[p]))
        else:
            sw = s[q] < s[p]
        s[p], s[q] = jnp.where(sw, s[q], s[p]), jnp.where(sw, s[p], s[q])
        ii[p], ii[q] = jnp.where(sw, ii[q], ii[p]), jnp.where(sw, ii[p], ii[q])

    cmpex(0, 1, False)
    cmpex(2, 3, False)
    cmpex(0, 2, False)
    cmpex(1, 3, False)
    cmpex(1, 2, True)

    inf = jnp.float32(jnp.inf)
    big = jnp.float32(2 * N)
    outs = []
    for t in range(K + 1):
        m = jnp.min(s[0], axis=1, keepdims=True)
        cand = jnp.where(s[0] == m, ii[0], big)
        sel = jnp.min(cand, axis=1, keepdims=True)
        if t > 0:
            outs.append(sel)
        if t < K:
            ext = cand == sel
            s[0] = jnp.where(ext, s[1], s[0])
            s[1] = jnp.where(ext, s[2], s[1])
            s[2] = jnp.where(ext, s[3], s[2])
            s[3] = jnp.where(ext, inf, s[3])
            ii[0] = jnp.where(ext, ii[1], ii[0])
            ii[1] = jnp.where(ext, ii[2], ii[1])
            ii[2] = jnp.where(ext, ii[3], ii[2])
    idx = jnp.concatenate(outs, axis=1).astype(jnp.int32)
    idx_ref[0] = idx + b * N


def _knn_indices(xyz, b):
    return pl.pallas_call(
        functools.partial(_topk_body, b=b),
        grid=(N // _ROWS,),
        in_specs=[
            pl.BlockSpec((1, _ROWS, 3), lambda i, b=b: (b, i, 0)),
            pl.BlockSpec((1, N, 3), lambda i, b=b: (b, 0, 0)),
        ],
        out_specs=pl.BlockSpec((1, _ROWS, K), lambda i: (0, i, 0)),
        out_shape=jax.ShapeDtypeStruct((1, N, K), jnp.int32),
    )(xyz, xyz)


_NC, _NS = 2, 16
_NW = _NC * _NS
_TOT = N * K
_PER_W = _TOT // _NW
_CH = 128
_NCH = _PER_W // _CH


def _gather_body(feat_hbm, idx_hbm, out_hbm, idx_v, rows0, rows1, sem0, sem1):
    wid = lax.axis_index("s") * _NC + lax.axis_index("c")
    pltpu.sync_copy(idx_hbm.at[pl.ds(wid * _NCH, _NCH)], idx_v)
    rows = (rows0, rows1)
    sems = (sem0, sem1)
    inflight = pltpu.async_copy(feat_hbm.at[idx_v.at[0]], rows[0], sems[0])
    for j in range(_NCH):
        cur = inflight
        if j + 1 < _NCH:
            inflight = pltpu.async_copy(
                feat_hbm.at[idx_v.at[j + 1]], rows[(j + 1) % 2], sems[(j + 1) % 2]
            )
        cur.wait()
        pltpu.sync_copy(rows[j % 2], out_hbm.at[pl.ds(wid * _PER_W + j * _CH, _CH)])


@functools.cache
def _make_sc_gather():
    mesh = plsc.VectorSubcoreMesh(
        core_axis_name="c", subcore_axis_name="s", num_cores=_NC, num_subcores=_NS
    )
    return pl.kernel(
        _gather_body,
        out_type=jax.ShapeDtypeStruct((_TOT, D), jnp.float32),
        mesh=mesh,
        scratch_types=[
            pltpu.VMEM((_NCH, _CH), jnp.int32),
            pltpu.VMEM((_CH, D), jnp.float32),
            pltpu.VMEM((_CH, D), jnp.float32),
            pltpu.SemaphoreType.DMA,
            pltpu.SemaphoreType.DMA,
        ],
    )


def _sc_gather(feat_b, idx2d):
    return _make_sc_gather()(feat_b, idx2d)


_QR = 256


def _attn_body(q_ref, g_ref, o_ref):
    q = q_ref[...]
    g = g_ref[...]
    s = jnp.sum(g * q[:, None, :], axis=2) * SCALE
    m = jnp.max(s, axis=1, keepdims=True)
    e = jnp.exp(s - m)
    w = e / jnp.sum(e, axis=1, keepdims=True)
    o_ref[...] = jnp.sum(g * w[:, :, None], axis=1)


def _attention(feat_flat, gathered, b):
    nblk = N // _QR
    return pl.pallas_call(
        _attn_body,
        grid=(nblk,),
        in_specs=[
            pl.BlockSpec((_QR, D), lambda i, b=b: (b * nblk + i, 0)),
            pl.BlockSpec((_QR, K, D), lambda i: (i, 0, 0)),
        ],
        out_specs=pl.BlockSpec((_QR, D), lambda i: (i, 0)),
        out_shape=jax.ShapeDtypeStruct((N, D), jnp.float32),
    )(feat_flat, gathered)


def kernel(xyz, features):
    feat_flat = features.reshape(B * N, D)
    outs = []
    for b in range(B):
        idx = _knn_indices(xyz, b)
        idx2d = idx.reshape(_TOT // _CH, _CH)
        gathered = _sc_gather(feat_flat, idx2d).reshape(N, K, D)
        outs.append(_attention(feat_flat, gathered, b))
    return jnp.stack(outs, axis=0)


# speedup vs baseline: 1.0501x; 1.0501x over previous
import functools

import jax
import jax.numpy as jnp
from jax import lax
from jax.experimental import pallas as pl
from jax.experimental.pallas import tpu as pltpu
from jax.experimental.pallas import tpu_sc as plsc

B = 4
N = 2048
D = 256
K = 16
SCALE = 0.0625


_ROWS = 256


def _topk_body(xyz_blk_ref, xyz_all_ref, idx_ref, *, b):
    x = xyz_blk_ref[0]
    y = xyz_all_ref[0]
    sqx = jnp.sum(x * x, axis=1)
    sqy = jnp.sum(y * y, axis=1)
    dot = lax.dot_general(
        x, y, (((1,), (1,)), ((), ())),
        preferred_element_type=jnp.float32,
    )
    d2 = (sqx[:, None] + sqy[None, :]) - 2.0 * dot
    vals = jnp.sqrt(jnp.maximum(d2, 0.0))
    iota = lax.broadcasted_iota(jnp.int32, (_ROWS, N), 1).astype(jnp.float32)
    inf = jnp.float32(jnp.inf)
    big = jnp.float32(N)
    outs = []
    for t in range(K + 1):
        m = jnp.min(vals, axis=1, keepdims=True)
        cand = jnp.where(vals == m, iota, big)
        sel = jnp.min(cand, axis=1, keepdims=True)
        if t > 0:
            outs.append(sel)
        if t < K:
            vals = jnp.where(cand == sel, inf, vals)
    idx = jnp.concatenate(outs, axis=1).astype(jnp.int32)
    idx_ref[0] = jnp.transpose(idx) + b * N


def _knn_indices(xyz, b):
    return pl.pallas_call(
        functools.partial(_topk_body, b=b),
        grid=(N // _ROWS,),
        in_specs=[
            pl.BlockSpec((1, _ROWS, 3), lambda i, b=b: (b, i, 0)),
            pl.BlockSpec((1, N, 3), lambda i, b=b: (b, 0, 0)),
        ],
        out_specs=pl.BlockSpec((1, K, _ROWS), lambda i: (0, 0, i)),
        out_shape=jax.ShapeDtypeStruct((1, K, N), jnp.int32),
    )(xyz, xyz)


_NC, _NS = 2, 16
_NW = _NC * _NS
_TOT = N * K
_PER_W = _TOT // _NW
_CH = 128
_NCH = _PER_W // _CH


def _gather_body(feat_hbm, idx_hbm, out_hbm, idx_v, rows0, rows1, sem0, sem1):
    wid = lax.axis_index("s") * _NC + lax.axis_index("c")
    pltpu.sync_copy(idx_hbm.at[pl.ds(wid * _NCH, _NCH)], idx_v)
    rows = (rows0, rows1)
    sems = (sem0, sem1)
    inflight = pltpu.async_copy(feat_hbm.at[idx_v.at[0]], rows[0], sems[0])
    for j in range(_NCH):
        cur = inflight
        if j + 1 < _NCH:
            inflight = pltpu.async_copy(
                feat_hbm.at[idx_v.at[j + 1]], rows[(j + 1) % 2], sems[(j + 1) % 2]
            )
        cur.wait()
        pltpu.sync_copy(rows[j % 2], out_hbm.at[pl.ds(wid * _PER_W + j * _CH, _CH)])


@functools.cache
def _make_sc_gather():
    mesh = plsc.VectorSubcoreMesh(
        core_axis_name="c", subcore_axis_name="s", num_cores=_NC, num_subcores=_NS
    )
    return pl.kernel(
        _gather_body,
        out_type=jax.ShapeDtypeStruct((_TOT, D), jnp.float32),
        mesh=mesh,
        scratch_types=[
            pltpu.VMEM((_NCH, _CH), jnp.int32),
            pltpu.VMEM((_CH, D), jnp.float32),
            pltpu.VMEM((_CH, D), jnp.float32),
            pltpu.SemaphoreType.DMA,
            pltpu.SemaphoreType.DMA,
        ],
    )


def _sc_gather(feat_b, idx2d):
    return _make_sc_gather()(feat_b, idx2d)


_QR = 256


def _attn_body(q_ref, g_ref, o_ref):
    q = q_ref[...]
    g = g_ref[...]
    s = jnp.sum(g * q[None, :, :], axis=2) * SCALE
    m = jnp.max(s, axis=0, keepdims=True)
    e = jnp.exp(s - m)
    w = e / jnp.sum(e, axis=0, keepdims=True)
    o_ref[...] = jnp.sum(g * w[:, :, None], axis=0)


def _attention(feat_flat, gathered, b):
    nblk = N // _QR
    return pl.pallas_call(
        _attn_body,
        grid=(nblk,),
        in_specs=[
            pl.BlockSpec((_QR, D), lambda i, b=b: (b * nblk + i, 0)),
            pl.BlockSpec((K, _QR, D), lambda i: (0, i, 0)),
        ],
        out_specs=pl.BlockSpec((_QR, D), lambda i: (i, 0)),
        out_shape=jax.ShapeDtypeStruct((N, D), jnp.float32),
    )(feat_flat, gathered)


def kernel(xyz, features):
    feat_flat = features.reshape(B * N, D)
    outs = []
    for b in range(B):
        idx = _knn_indices(xyz, b)
        idx2d = idx.reshape(_TOT // _CH, _CH)
        gathered = _sc_gather(feat_flat, idx2d).reshape(K, N, D)
        outs.append(_attention(feat_flat, gathered, b))
    return jnp.stack(outs, axis=0)
